# Initial kernel scaffold; baseline (speedup 1.0000x reference)
import functools

import jax
import jax.numpy as jnp
from jax.experimental import pallas as pl

K = 10
N = 1024
B = 8
BN_EPS = 1e-5


def _lrelu(x):
    return jnp.where(x >= 0, x, 0.2 * x)


def _finalize_stats(s1, s2, count):
    m = jnp.sum(s1[:, 0, :], axis=0, keepdims=True) / count
    ex2 = jnp.sum(s2[:, 0, :], axis=0, keepdims=True) / count
    var = ex2 - m * m
    return m, jax.lax.rsqrt(var + BN_EPS)


def _knn_accumulate(x, u):
    f32 = jnp.float32
    col = jnp.sum(x * x, axis=1, keepdims=True)
    row = jax.lax.dot_general(jnp.ones((1, x.shape[1]), f32), x * x,
                              (((1,), (1,)), ((), ())),
                              preferred_element_type=f32)
    cross = jax.lax.dot_general(x, x, (((1,), (1,)), ((), ())),
                                preferred_element_type=f32)
    d = col + row - 2.0 * cross
    d = jnp.maximum(d, 1e-12)
    i_n = jax.lax.broadcasted_iota(jnp.int32, (N, N), 0)
    i_m = jax.lax.broadcasted_iota(jnp.int32, (N, N), 1)
    d = jnp.where(i_n == i_m, jnp.inf, d)

    c_out = u.shape[1]
    gmax = jnp.full((N, c_out), -jnp.inf, f32)
    gsum = jnp.zeros((N, c_out), f32)
    gsq = jnp.zeros((N, c_out), f32)
    for _ in range(K):
        m1 = jnp.min(d, axis=1, keepdims=True)
        cand = jnp.where(d == m1, i_m, N)
        idx = jnp.min(cand, axis=1, keepdims=True)
        sel = i_m == idx
        g = jax.lax.dot_general(sel.astype(f32), u,
                                (((1,), (0,)), ((), ())),
                                preferred_element_type=f32)
        gmax = jnp.maximum(gmax, g)
        gsum = gsum + g
        gsq = gsq + g * g
        d = jnp.where(sel, jnp.inf, d)
    return gmax, gsum, gsq


def _block_body(first, mprev_ref, s1_ref, s2_ref, g_ref, b_ref,
                uw_ref, vw_ref, m_out_ref, s1_out_ref, s2_out_ref):
    f32 = jnp.float32
    if first:
        x = mprev_ref[0]
    else:
        m, rs = _finalize_stats(s1_ref[...], s2_ref[...], float(B * N * K))
        x = _lrelu((mprev_ref[0] - m) * rs * g_ref[...] + b_ref[...])
    u = jax.lax.dot_general(x, uw_ref[...], (((1,), (0,)), ((), ())),
                            preferred_element_type=f32)
    v = jax.lax.dot_general(x, vw_ref[...], (((1,), (0,)), ((), ())),
                            preferred_element_type=f32)
    gmax, gsum, gsq = _knn_accumulate(x, u)
    m_out_ref[0] = gmax + v
    s1_out_ref[0] = jnp.sum(gsum + K * v, axis=0, keepdims=True)
    s2_out_ref[0] = jnp.sum(gsq + 2.0 * v * gsum + K * (v * v),
                            axis=0, keepdims=True)


def _edge_block(first, c_in, c_out, mprev, s1, s2, g, b, uw, vw):
    f32 = jnp.float32

    def full(*dims):
        return pl.BlockSpec(dims, lambda i: (0,) * len(dims))

    return pl.pallas_call(
        functools.partial(_block_body, first),
        grid=(B,),
        in_specs=[
            pl.BlockSpec((1, N, c_in), lambda i: (i, 0, 0)),
            full(B, 1, c_in),
            full(B, 1, c_in),
            full(1, c_in),
            full(1, c_in),
            full(c_in, c_out),
            full(c_in, c_out),
        ],
        out_specs=[
            pl.BlockSpec((1, N, c_out), lambda i: (i, 0, 0)),
            pl.BlockSpec((1, 1, c_out), lambda i: (i, 0, 0)),
            pl.BlockSpec((1, 1, c_out), lambda i: (i, 0, 0)),
        ],
        out_shape=[
            jax.ShapeDtypeStruct((B, N, c_out), f32),
            jax.ShapeDtypeStruct((B, 1, c_out), f32),
            jax.ShapeDtypeStruct((B, 1, c_out), f32),
        ],
    )(mprev, s1, s2, g, b, uw, vw)


def _conv5_body(m1_ref, s11, s21, g1, b1, m2_ref, s12, s22, g2, b2,
                m3_ref, s13, s23, g3, b3, m4_ref, s14, s24, g4, b4,
                w5_ref, m5_out, s1_out, s2_out):
    f32 = jnp.float32
    cnt = float(B * N * K)
    xs = []
    for mr, s1r, s2r, gr, br in ((m1_ref, s11, s21, g1, b1),
                                 (m2_ref, s12, s22, g2, b2),
                                 (m3_ref, s13, s23, g3, b3),
                                 (m4_ref, s14, s24, g4, b4)):
        m, rs = _finalize_stats(s1r[...], s2r[...], cnt)
        xs.append(_lrelu((mr[0] - m) * rs * gr[...] + br[...]))
    x = jnp.concatenate(xs, axis=1)
    y = jax.lax.dot_general(x, w5_ref[...], (((1,), (0,)), ((), ())),
                            preferred_element_type=f32)
    m5_out[0] = jnp.max(y, axis=0, keepdims=True)
    s1_out[0] = jnp.sum(y, axis=0, keepdims=True)
    s2_out[0] = jnp.sum(y * y, axis=0, keepdims=True)


def _head_body(m5_ref, s1_ref, s2_ref, g5, b5, wf1, gf1, bf1,
               wf2, bf2, gf2, bf2n, wf3, bf3, out_ref):
    f32 = jnp.float32
    m, rs = _finalize_stats(s1_ref[...], s2_ref[...], float(B * N))
    o = _lrelu((m5_ref[:, 0, :] - m) * rs * g5[...] + b5[...])
    h = jax.lax.dot_general(o, wf1[...], (((1,), (0,)), ((), ())),
                            preferred_element_type=f32)
    hm = jnp.mean(h, axis=0, keepdims=True)
    hc = h - hm
    hv = jnp.mean(hc * hc, axis=0, keepdims=True)
    h = _lrelu(hc * jax.lax.rsqrt(hv + BN_EPS) * gf1[...] + bf1[...])
    h = jax.lax.dot_general(h, wf2[...], (((1,), (0,)), ((), ())),
                            preferred_element_type=f32) + bf2[...]
    hm = jnp.mean(h, axis=0, keepdims=True)
    hc = h - hm
    hv = jnp.mean(hc * hc, axis=0, keepdims=True)
    h = _lrelu(hc * jax.lax.rsqrt(hv + BN_EPS) * gf2[...] + bf2n[...])
    out_ref[...] = jax.lax.dot_general(h, wf3[...], (((1,), (0,)), ((), ())),
                                       preferred_element_type=f32) + bf3[...]


def kernel(points, Wc1, g1, b1, Wc2, g2, b2, Wc3, g3, b3, Wc4, g4, b4,
           Wc5, g5, b5, Wf1, gf1, bf1, Wf2, bf2, gf2, bf2n, Wf3, bf3):
    f32 = jnp.float32

    def split_w(w, c_in):
        w1 = w[:, :c_in]
        w2 = w[:, c_in:]
        return w1.T, (w2 - w1).T

    def r2(a):
        return a.reshape(1, -1).astype(f32)

    zero_s = jnp.zeros((B, 1, 3), f32)
    zero_g = jnp.zeros((1, 3), f32)

    uw1, vw1 = split_w(Wc1, 3)
    m1, s11, s21 = _edge_block(True, 3, 64, points, zero_s, zero_s,
                               zero_g, zero_g, uw1, vw1)
    uw2, vw2 = split_w(Wc2, 64)
    m2, s12, s22 = _edge_block(False, 64, 64, m1, s11, s21,
                               r2(g1), r2(b1), uw2, vw2)
    uw3, vw3 = split_w(Wc3, 64)
    m3, s13, s23 = _edge_block(False, 64, 128, m2, s12, s22,
                               r2(g2), r2(b2), uw3, vw3)
    uw4, vw4 = split_w(Wc4, 128)
    m4, s14, s24 = _edge_block(False, 128, 256, m3, s13, s23,
                               r2(g3), r2(b3), uw4, vw4)

    def full(*dims):
        return pl.BlockSpec(dims, lambda i: (0,) * len(dims))

    m5, s15, s25 = pl.pallas_call(
        _conv5_body,
        grid=(B,),
        in_specs=[
            pl.BlockSpec((1, N, 64), lambda i: (i, 0, 0)),
            full(B, 1, 64), full(B, 1, 64), full(1, 64), full(1, 64),
            pl.BlockSpec((1, N, 64), lambda i: (i, 0, 0)),
            full(B, 1, 64), full(B, 1, 64), full(1, 64), full(1, 64),
            pl.BlockSpec((1, N, 128), lambda i: (i, 0, 0)),
            full(B, 1, 128), full(B, 1, 128), full(1, 128), full(1, 128),
            pl.BlockSpec((1, N, 256), lambda i: (i, 0, 0)),
            full(B, 1, 256), full(B, 1, 256), full(1, 256), full(1, 256),
            full(512, 1024),
        ],
        out_specs=[
            pl.BlockSpec((1, 1, 1024), lambda i: (i, 0, 0)),
            pl.BlockSpec((1, 1, 1024), lambda i: (i, 0, 0)),
            pl.BlockSpec((1, 1, 1024), lambda i: (i, 0, 0)),
        ],
        out_shape=[
            jax.ShapeDtypeStruct((B, 1, 1024), f32),
            jax.ShapeDtypeStruct((B, 1, 1024), f32),
            jax.ShapeDtypeStruct((B, 1, 1024), f32),
        ],
    )(m1, s11, s21, r2(g1), r2(b1),
      m2, s12, s22, r2(g2), r2(b2),
      m3, s13, s23, r2(g3), r2(b3),
      m4, s14, s24, r2(g4), r2(b4),
      Wc5.T)

    out = pl.pallas_call(
        _head_body,
        out_shape=jax.ShapeDtypeStruct((B, 3), f32),
    )(m5, s15, s25, r2(g5), r2(b5), Wf1.T, r2(gf1), r2(bf1),
      Wf2.T, r2(bf2), r2(gf2), r2(bf2n), Wf3.T, r2(bf3))
    return out


# speedup vs baseline: 2.0759x; 2.0759x over previous
import functools

import jax
import jax.numpy as jnp
from jax.experimental import pallas as pl

K = 10
N = 1024
B = 8
BN_EPS = 1e-5


def _lrelu(x):
    return jnp.where(x >= 0, x, 0.2 * x)


def _bn(x, g, b):
    axes = tuple(i for i in range(x.ndim) if i != 1)
    m = jnp.mean(x, axis=axes, keepdims=True)
    v = jnp.var(x, axis=axes, keepdims=True)
    shp = [1] * x.ndim
    shp[1] = -1
    return g.reshape(shp) * (x - m) * jax.lax.rsqrt(v + BN_EPS) + b.reshape(shp)


def _knn_body(x_ref, d2c_ref, d2r_ref, idx_out):
    f32 = jnp.float32
    xx = x_ref[0]
    cross = jax.lax.dot_general(xx, xx, (((1,), (1,)), ((), ())),
                                preferred_element_type=f32)
    d = jnp.sqrt(jnp.maximum(d2c_ref[0] + d2r_ref[0] - 2.0 * cross, 1e-12))
    i_m = jax.lax.broadcasted_iota(jnp.int32, (N, N), 1)
    for j in range(K + 1):
        m1 = jnp.min(d, axis=1, keepdims=True)
        cand = jnp.where(d == m1, i_m, N)
        idx = jnp.min(cand, axis=1, keepdims=True)
        sel = i_m == idx
        d = jnp.where(sel, jnp.inf, d)
        if j > 0:
            idx_out[0, :, j - 1] = idx[:, 0]


def _knn_inds(c_in, x, d2):
    return pl.pallas_call(
        _knn_body,
        grid=(B,),
        in_specs=[
            pl.BlockSpec((1, N, c_in), lambda i: (i, 0, 0)),
            pl.BlockSpec((1, N, 1), lambda i: (i, 0, 0)),
            pl.BlockSpec((1, 1, N), lambda i: (i, 0, 0)),
        ],
        out_specs=pl.BlockSpec((1, N, K), lambda i: (i, 0, 0)),
        out_shape=jax.ShapeDtypeStruct((B, N, K), jnp.int32),
    )(x, d2.reshape(B, N, 1), d2.reshape(B, 1, N))


def _conv5_body(x_ref, w_ref, m_out, s1_out, s2_out):
    f32 = jnp.float32
    y = jax.lax.dot_general(x_ref[0], w_ref[...], (((1,), (1,)), ((), ())),
                            preferred_element_type=f32)
    m_out[0] = jnp.max(y, axis=0, keepdims=True)
    s1_out[0] = jnp.sum(y, axis=0, keepdims=True)
    s2_out[0] = jnp.sum(y * y, axis=0, keepdims=True)


def _head_body(m5_ref, s1_ref, s2_ref, g5, b5, wf1, gf1, bf1,
               wf2, bf2, gf2, bf2n, wf3, bf3, out_ref):
    f32 = jnp.float32
    cnt = float(B * N)
    m = jnp.sum(s1_ref[:, 0, :], axis=0, keepdims=True) / cnt
    ex2 = jnp.sum(s2_ref[:, 0, :], axis=0, keepdims=True) / cnt
    rs = jax.lax.rsqrt(ex2 - m * m + BN_EPS)
    o = _lrelu(g5[...] * (m5_ref[:, 0, :] - m) * rs + b5[...])

    h = jax.lax.dot_general(o, wf1[...], (((1,), (1,)), ((), ())),
                            preferred_element_type=f32)
    hm = jnp.mean(h, axis=0, keepdims=True)
    hc = h - hm
    hv = jnp.mean(hc * hc, axis=0, keepdims=True)
    h = _lrelu(gf1[...] * hc * jax.lax.rsqrt(hv + BN_EPS) + bf1[...])
    h = jax.lax.dot_general(h, wf2[...], (((1,), (1,)), ((), ())),
                            preferred_element_type=f32) + bf2[...]
    hm = jnp.mean(h, axis=0, keepdims=True)
    hc = h - hm
    hv = jnp.mean(hc * hc, axis=0, keepdims=True)
    h = _lrelu(gf2[...] * hc * jax.lax.rsqrt(hv + BN_EPS) + bf2n[...])
    out_ref[...] = jax.lax.dot_general(h, wf3[...], (((1,), (1,)), ((), ())),
                                       preferred_element_type=f32) + bf3[...]


def kernel(points, Wc1, g1, b1, Wc2, g2, b2, Wc3, g3, b3, Wc4, g4, b4,
           Wc5, g5, b5, Wf1, gf1, bf1, Wf2, bf2, gf2, bf2n, Wf3, bf3):
    f32 = jnp.float32

    def r2(a):
        return a.reshape(1, -1).astype(f32)

    x = points
    outs = []
    for w, g, bb, c_in in ((Wc1, g1, b1, 3), (Wc2, g2, b2, 64),
                           (Wc3, g3, b3, 64), (Wc4, g4, b4, 128)):
        d2 = jnp.sum(x * x, axis=-1)
        inds = _knn_inds(c_in, x, d2)
        feats = jax.vmap(lambda xb, ib: xb[ib])(x, inds)
        xk = jnp.broadcast_to(x[:, :, None, :], feats.shape)
        e = jnp.transpose(jnp.concatenate([feats - xk, xk], axis=-1),
                          (0, 3, 1, 2))
        y = jnp.einsum('oc,bc...->bo...', w, e)
        out = jnp.max(_lrelu(_bn(y, g, bb)), axis=-1)
        outs.append(out)
        x = jnp.transpose(out, (0, 2, 1))

    cat = jnp.concatenate(outs, axis=1)
    x5 = jnp.transpose(cat, (0, 2, 1))
    m5, s15, s25 = pl.pallas_call(
        _conv5_body,
        grid=(B,),
        in_specs=[pl.BlockSpec((1, N, 512), lambda i: (i, 0, 0)),
                  pl.BlockSpec((1024, 512), lambda i: (0, 0))],
        out_specs=[pl.BlockSpec((1, 1, 1024), lambda i: (i, 0, 0))] * 3,
        out_shape=[jax.ShapeDtypeStruct((B, 1, 1024), f32)] * 3,
    )(x5, Wc5)

    out = pl.pallas_call(
        _head_body,
        out_shape=jax.ShapeDtypeStruct((B, 3), f32),
    )(m5, s15, s25, r2(g5), r2(b5), Wf1, r2(gf1), r2(bf1),
      Wf2, r2(bf2), r2(gf2), r2(bf2n), Wf3, r2(bf3))
    return out


# speedup vs baseline: 5.0125x; 2.4146x over previous
import functools

import jax
import jax.numpy as jnp
from jax.experimental import pallas as pl

K = 10
N = 1024
B = 8
BN_EPS = 1e-5


def _lrelu(x):
    return jnp.where(x >= 0, x, 0.2 * x)


def _bn(x, g, b):
    axes = tuple(i for i in range(x.ndim) if i != 1)
    m = jnp.mean(x, axis=axes, keepdims=True)
    v = jnp.var(x, axis=axes, keepdims=True)
    shp = [1] * x.ndim
    shp[1] = -1
    return g.reshape(shp) * (x - m) * jax.lax.rsqrt(v + BN_EPS) + b.reshape(shp)


def _select_edges(xx, d2c, d2r):
    f32 = jnp.float32
    cross = jax.lax.dot_general(xx, xx, (((1,), (1,)), ((), ())),
                                preferred_element_type=f32)
    d = jnp.sqrt(jnp.maximum(d2c + d2r - 2.0 * cross, 1e-12))
    i_m = jax.lax.broadcasted_iota(jnp.int32, (N, N), 1)
    gs = []
    for j in range(K + 1):
        m1 = jnp.min(d, axis=1, keepdims=True)
        cand = jnp.where(d == m1, i_m, N)
        idx = jnp.min(cand, axis=1, keepdims=True)
        sel = i_m == idx
        d = jnp.where(sel, jnp.inf, d)
        if j > 0:
            gs.append(jax.lax.dot_general(
                sel.astype(f32), xx, (((1,), (0,)), ((), ())),
                precision=jax.lax.Precision.HIGHEST,
                preferred_element_type=f32))
    return gs


def _knn_edges_body(x_ref, d2c_ref, d2r_ref, e_out):
    xx = x_ref[0]
    gs = _select_edges(xx, d2c_ref[0], d2r_ref[0])
    es = [jnp.concatenate([g - xx, xx], axis=1) for g in gs]
    e_out[0] = jnp.concatenate(es, axis=0)


def _knn_edges(c_in, x, d2):
    return pl.pallas_call(
        _knn_edges_body,
        grid=(B,),
        in_specs=[
            pl.BlockSpec((1, N, c_in), lambda i: (i, 0, 0)),
            pl.BlockSpec((1, N, 1), lambda i: (i, 0, 0)),
            pl.BlockSpec((1, 1, N), lambda i: (i, 0, 0)),
        ],
        out_specs=pl.BlockSpec((1, K * N, 2 * c_in), lambda i: (i, 0, 0)),
        out_shape=jax.ShapeDtypeStruct((B, K * N, 2 * c_in), jnp.float32),
    )(x, d2.reshape(B, N, 1), d2.reshape(B, 1, N))


def _edge4_body(x_ref, d2c_ref, d2r_ref, w_ref, m_out, s1_out, s2_out):
    f32 = jnp.float32
    xx = x_ref[0]
    gs = _select_edges(xx, d2c_ref[0], d2r_ref[0])
    es = [jnp.concatenate([g - xx, xx], axis=1) for g in gs]
    e2d = jnp.concatenate(es, axis=0)
    yt = jax.lax.dot_general(w_ref[...], e2d, (((1,), (1,)), ((), ())),
                             preferred_element_type=f32)
    mx = yt[:, :N]
    for j in range(1, K):
        mx = jnp.maximum(mx, yt[:, j * N:(j + 1) * N])
    m_out[0] = mx
    s1_out[0] = jnp.sum(yt, axis=1, keepdims=True)
    s2_out[0] = jnp.sum(yt * yt, axis=1, keepdims=True)


def _edge4_block(c_in, c_out, x, d2, w):
    f32 = jnp.float32
    return pl.pallas_call(
        _edge4_body,
        grid=(B,),
        in_specs=[
            pl.BlockSpec((1, N, c_in), lambda i: (i, 0, 0)),
            pl.BlockSpec((1, N, 1), lambda i: (i, 0, 0)),
            pl.BlockSpec((1, 1, N), lambda i: (i, 0, 0)),
            pl.BlockSpec((c_out, 2 * c_in), lambda i: (0, 0)),
        ],
        out_specs=[
            pl.BlockSpec((1, c_out, N), lambda i: (i, 0, 0)),
            pl.BlockSpec((1, c_out, 1), lambda i: (i, 0, 0)),
            pl.BlockSpec((1, c_out, 1), lambda i: (i, 0, 0)),
        ],
        out_shape=[
            jax.ShapeDtypeStruct((B, c_out, N), f32),
            jax.ShapeDtypeStruct((B, c_out, 1), f32),
            jax.ShapeDtypeStruct((B, c_out, 1), f32),
        ],
    )(x, d2.reshape(B, N, 1), d2.reshape(B, 1, N), w)


def _conv5_body(x_ref, w_ref, m_out, s1_out, s2_out):
    f32 = jnp.float32
    y = jax.lax.dot_general(x_ref[0], w_ref[...], (((1,), (1,)), ((), ())),
                            preferred_element_type=f32)
    m_out[0] = jnp.max(y, axis=0, keepdims=True)
    s1_out[0] = jnp.sum(y, axis=0, keepdims=True)
    s2_out[0] = jnp.sum(y * y, axis=0, keepdims=True)


def _head_body(m5_ref, s1_ref, s2_ref, g5, b5, wf1, gf1, bf1,
               wf2, bf2, gf2, bf2n, wf3, bf3, out_ref):
    f32 = jnp.float32
    cnt = float(B * N)
    m = jnp.sum(s1_ref[:, 0, :], axis=0, keepdims=True) / cnt
    ex2 = jnp.sum(s2_ref[:, 0, :], axis=0, keepdims=True) / cnt
    rs = jax.lax.rsqrt(ex2 - m * m + BN_EPS)
    o = _lrelu(g5[...] * (m5_ref[:, 0, :] - m) * rs + b5[...])

    h = jax.lax.dot_general(o, wf1[...], (((1,), (1,)), ((), ())),
                            preferred_element_type=f32)
    hm = jnp.mean(h, axis=0, keepdims=True)
    hc = h - hm
    hv = jnp.mean(hc * hc, axis=0, keepdims=True)
    h = _lrelu(gf1[...] * hc * jax.lax.rsqrt(hv + BN_EPS) + bf1[...])
    h = jax.lax.dot_general(h, wf2[...], (((1,), (1,)), ((), ())),
                            preferred_element_type=f32) + bf2[...]
    hm = jnp.mean(h, axis=0, keepdims=True)
    hc = h - hm
    hv = jnp.mean(hc * hc, axis=0, keepdims=True)
    h = _lrelu(gf2[...] * hc * jax.lax.rsqrt(hv + BN_EPS) + bf2n[...])
    out_ref[...] = jax.lax.dot_general(h, wf3[...], (((1,), (1,)), ((), ())),
                                       preferred_element_type=f32) + bf3[...]


def kernel(points, Wc1, g1, b1, Wc2, g2, b2, Wc3, g3, b3, Wc4, g4, b4,
           Wc5, g5, b5, Wf1, gf1, bf1, Wf2, bf2, gf2, bf2n, Wf3, bf3):
    f32 = jnp.float32

    def r2(a):
        return a.reshape(1, -1).astype(f32)

    x = points
    outs = []
    for w, g, bb, c_in in ((Wc1, g1, b1, 3), (Wc2, g2, b2, 64),
                           (Wc3, g3, b3, 64)):
        d2 = jnp.sum(x * x, axis=-1)
        eflat = _knn_edges(c_in, x, d2)
        e = jnp.transpose(eflat.reshape(B, K, N, 2 * c_in),
                          (0, 3, 2, 1))
        y = jnp.einsum('oc,bc...->bo...', w, e)
        out = jnp.max(_lrelu(_bn(y, g, bb)), axis=-1)
        outs.append(out)
        x = jnp.transpose(out, (0, 2, 1))

    d2 = jnp.sum(x * x, axis=-1)
    m4x, s14, s24 = _edge4_block(128, 256, x, d2, Wc4)
    cnt4 = float(B * N * K)
    m4 = jnp.sum(s14, axis=0, keepdims=True) / cnt4
    v4 = jnp.sum(s24, axis=0, keepdims=True) / cnt4 - m4 * m4
    shp = (1, -1, 1)
    o4 = (g4.reshape(shp) * (m4x - m4) * jax.lax.rsqrt(v4 + BN_EPS)
          + b4.reshape(shp))
    outs.append(_lrelu(o4))

    cat = jnp.concatenate(outs, axis=1)
    x5 = jnp.transpose(cat, (0, 2, 1))
    m5, s15, s25 = pl.pallas_call(
        _conv5_body,
        grid=(B,),
        in_specs=[pl.BlockSpec((1, N, 512), lambda i: (i, 0, 0)),
                  pl.BlockSpec((1024, 512), lambda i: (0, 0))],
        out_specs=[pl.BlockSpec((1, 1, 1024), lambda i: (i, 0, 0))] * 3,
        out_shape=[jax.ShapeDtypeStruct((B, 1, 1024), f32)] * 3,
    )(x5, Wc5)

    out = pl.pallas_call(
        _head_body,
        out_shape=jax.ShapeDtypeStruct((B, 3), f32),
    )(m5, s15, s25, r2(g5), r2(b5), Wf1, r2(gf1), r2(bf1),
      Wf2, r2(bf2), r2(gf2), r2(bf2n), Wf3, r2(bf3))
    return out
